# baseline (speedup 1.0000x reference)
import functools

import jax
import jax.numpy as jnp
from jax import lax
from jax.experimental import pallas as pl
from jax.experimental.pallas import tpu as pltpu
from jax.experimental.pallas import tpu_sc as plsc

T = 32768
D = 256
B = 16
NC = 2
NS = 16
NW = NC * NS
RPW = T // NW
CHUNK = 16
NCH = RPW // CHUNK
L = 16

_mesh = plsc.VectorSubcoreMesh(
    core_axis_name="c", subcore_axis_name="s", num_cores=NC, num_subcores=NS)


@functools.partial(
    pl.kernel,
    out_type=jax.ShapeDtypeStruct((NC, B, D), jnp.float32),
    mesh=_mesh,
    scratch_types=[
        pltpu.VMEM((2, CHUNK, D), jnp.float32),
        pltpu.VMEM((RPW,), jnp.int32),
        pltpu.VMEM((B * D,), jnp.float32),
        pltpu.VMEM((NS, D), jnp.float32),
        pltpu.VMEM_SHARED((NS, B * D), jnp.float32),
        pltpu.SemaphoreType.DMA,
        pltpu.SemaphoreType.DMA,
    ],
    compiler_params=pltpu.CompilerParams(needs_layout_passes=False),
)
def _segsum_sc(flat_hbm, segs_hbm, out_hbm, buf, segs_v, acc_v, rows_v,
               shared, lsem0, lsem1):
  cid = lax.axis_index("c")
  sid = lax.axis_index("s")
  wid = sid * NC + cid
  base = wid * RPW
  lsems = (lsem0, lsem1)

  pltpu.sync_copy(segs_hbm.at[wid], segs_v)

  pltpu.async_copy(flat_hbm.at[pl.ds(base, CHUNK)], buf.at[0], lsem0)
  pltpu.async_copy(flat_hbm.at[pl.ds(base + CHUNK, CHUNK)], buf.at[1], lsem1)

  zeros16 = jnp.zeros((L,), jnp.float32)
  for j in range(B * D // L):
    acc_v[pl.ds(j * L, L)] = zeros16

  lane = lax.iota(jnp.int32, L)
  cols = [jnp.int32(j * L) + lane for j in range(D // L)]

  def accumulate(bufb, g):
    def group_step(g2, carry):
      row0 = 0
      segs16 = segs_v[pl.ds(g * CHUNK + row0, L)]
      mn = jnp.min(segs16)
      mx = jnp.max(segs16)

      @pl.when(mn == mx)
      def _uniform():
        base_idx = jnp.broadcast_to(mn, (L,)) * D
        for j in range(D // L):
          v = bufb[row0, pl.ds(j * L, L)]
          for i in range(1, L):
            v = v + bufb[row0 + i, pl.ds(j * L, L)]
          plsc.addupdate_scatter(acc_v, [base_idx + cols[j]], v)

      @pl.when(mn != mx)
      def _mixed():
        for i in range(L):
          row = row0 + i
          seg_b = plsc.load_gather(
              segs_v, [jnp.broadcast_to(g * CHUNK + row, (L,))])
          row_base = seg_b * D
          for j in range(D // L):
            vals = bufb[row, pl.ds(j * L, L)]
            plsc.addupdate_scatter(acc_v, [row_base + cols[j]], vals)
      return carry
    group_step(0, 0)

  def chunk_step(it, carry):
    for b in range(2):
      g = it * 2 + b
      row0 = base + (g + 2) * CHUNK
      pltpu.make_async_copy(
          flat_hbm.at[pl.ds(base, CHUNK)], buf.at[b], lsems[b]).wait()
      accumulate(buf.at[b], g)
      @pl.when(g + 2 < NCH)
      def _refill():
        pltpu.async_copy(flat_hbm.at[pl.ds(row0, CHUNK)], buf.at[b], lsems[b])
    return carry

  lax.fori_loop(0, NCH // 2, chunk_step, 0)

  pltpu.sync_copy(acc_v, shared.at[sid])
  plsc.subcore_barrier()

  for k in range(NS):
    pltpu.sync_copy(shared.at[k, pl.ds(sid * D, D)], rows_v.at[k])
  for j in range(D // L):
    sl = pl.ds(j * L, L)
    v = rows_v[0, sl]
    for k in range(1, NS):
      v = v + rows_v[k, sl]
    rows_v[0, sl] = v
  pltpu.sync_copy(rows_v.at[0], out_hbm.at[cid, sid])


def kernel(flat, segment_ids):
  segs = segment_ids.reshape(NW, RPW)
  partials = _segsum_sc(flat, segs)
  return partials[0] + partials[1]


# speedup vs baseline: 1.7582x; 1.7582x over previous
import functools

import jax
import jax.numpy as jnp
from jax import lax
from jax.experimental import pallas as pl
from jax.experimental.pallas import tpu as pltpu
from jax.experimental.pallas import tpu_sc as plsc

T = 32768
D = 256
B = 16
NC = 2
NS = 16
NW = NC * NS
SC_ROWS = 16384
TC_ROWS = T - SC_ROWS
RPW = SC_ROWS // NW
CHUNK = 128
NCH = RPW // CHUNK
L = 16
BLK = 2048

_mesh = plsc.VectorSubcoreMesh(
    core_axis_name="c", subcore_axis_name="s", num_cores=NC, num_subcores=NS)


@functools.partial(
    pl.kernel,
    out_type=jax.ShapeDtypeStruct((NC, B, D), jnp.float32),
    mesh=_mesh,
    scratch_types=[
        pltpu.VMEM((2, CHUNK, D), jnp.float32),
        pltpu.VMEM((RPW,), jnp.int32),
        pltpu.VMEM((B * D,), jnp.float32),
        pltpu.VMEM((NS, D), jnp.float32),
        pltpu.VMEM_SHARED((NS, B * D), jnp.float32),
        pltpu.SemaphoreType.DMA,
        pltpu.SemaphoreType.DMA,
    ],
    compiler_params=pltpu.CompilerParams(needs_layout_passes=False),
)
def _segsum_sc(flat_hbm, segs_hbm, out_hbm, buf, segs_v, acc_v, rows_v,
               shared, lsem0, lsem1):
  cid = lax.axis_index("c")
  sid = lax.axis_index("s")
  wid = sid * NC + cid
  base = wid * RPW
  lsems = (lsem0, lsem1)

  pltpu.sync_copy(segs_hbm.at[wid], segs_v)

  pltpu.async_copy(flat_hbm.at[pl.ds(base, CHUNK)], buf.at[0], lsem0)
  pltpu.async_copy(flat_hbm.at[pl.ds(base + CHUNK, CHUNK)], buf.at[1], lsem1)

  zeros16 = jnp.zeros((L,), jnp.float32)
  for j in range(B * D // L):
    acc_v[pl.ds(j * L, L)] = zeros16

  lane = lax.iota(jnp.int32, L)
  cols = [jnp.int32(j * L) + lane for j in range(D // L)]

  def accumulate(bufb, g):
    def group_step(g2, carry):
      row0 = g2 * L
      segs16 = segs_v[pl.ds(g * CHUNK + row0, L)]
      mn = jnp.min(segs16)
      mx = jnp.max(segs16)

      @pl.when(mn == mx)
      def _uniform():
        base_idx = jnp.broadcast_to(mn, (L,)) * D
        for j in range(D // L):
          v = bufb[row0, pl.ds(j * L, L)]
          for i in range(1, L):
            v = v + bufb[row0 + i, pl.ds(j * L, L)]
          plsc.addupdate_scatter(acc_v, [base_idx + cols[j]], v)

      @pl.when(mn != mx)
      def _mixed():
        for i in range(L):
          row = row0 + i
          seg_b = plsc.load_gather(
              segs_v, [jnp.broadcast_to(g * CHUNK + row, (L,))])
          row_base = seg_b * D
          for j in range(D // L):
            vals = bufb[row, pl.ds(j * L, L)]
            plsc.addupdate_scatter(acc_v, [row_base + cols[j]], vals)
      return carry
    lax.fori_loop(0, CHUNK // L, group_step, 0)

  def chunk_step(it, carry):
    for b in range(2):
      g = it * 2 + b
      row0 = base + (g + 2) * CHUNK
      pltpu.make_async_copy(
          flat_hbm.at[pl.ds(base, CHUNK)], buf.at[b], lsems[b]).wait()
      accumulate(buf.at[b], g)
      @pl.when(g + 2 < NCH)
      def _refill():
        pltpu.async_copy(flat_hbm.at[pl.ds(row0, CHUNK)], buf.at[b], lsems[b])
    return carry

  lax.fori_loop(0, NCH // 2, chunk_step, 0)

  pltpu.sync_copy(acc_v, shared.at[sid])
  plsc.subcore_barrier()

  for k in range(NS):
    pltpu.sync_copy(shared.at[k, pl.ds(sid * D, D)], rows_v.at[k])
  for j in range(D // L):
    sl = pl.ds(j * L, L)
    v = rows_v[0, sl]
    for k in range(1, NS):
      v = v + rows_v[k, sl]
    rows_v[0, sl] = v
  pltpu.sync_copy(rows_v.at[0], out_hbm.at[cid, sid])


def _segsum_tc_body(segs_ref, flat_ref, out_ref):
  @pl.when(pl.program_id(0) == 0)
  def _init():
    out_ref[...] = jnp.zeros_like(out_ref)
  seg_row = segs_ref[0, 0, :]
  onehot = (seg_row[None, :] == lax.iota(jnp.int32, B)[:, None]).astype(
      jnp.float32)
  out_ref[...] += jnp.dot(onehot, flat_ref[...],
                          preferred_element_type=jnp.float32,
                          precision=lax.Precision.HIGHEST)


_SC_BLKS = SC_ROWS // BLK

_segsum_tc = pl.pallas_call(
    _segsum_tc_body,
    grid=(TC_ROWS // BLK,),
    in_specs=[
        pl.BlockSpec((1, 1, BLK), lambda i: (_SC_BLKS + i, 0, 0)),
        pl.BlockSpec((BLK, D), lambda i: (_SC_BLKS + i, 0)),
    ],
    out_specs=pl.BlockSpec((B, D), lambda i: (0, 0)),
    out_shape=jax.ShapeDtypeStruct((B, D), jnp.float32),
)


def kernel(flat, segment_ids):
  segs_sc = segment_ids[:SC_ROWS].reshape(NW, RPW)
  segs_tc = segment_ids.reshape(T // BLK, 1, BLK)
  sc_part = _segsum_sc(flat, segs_sc)
  tc_part = _segsum_tc(segs_tc, flat)
  return sc_part[0] + sc_part[1] + tc_part


# speedup vs baseline: 1.9793x; 1.1258x over previous
import functools

import jax
import jax.numpy as jnp
from jax import lax
from jax.experimental import pallas as pl
from jax.experimental.pallas import tpu as pltpu
from jax.experimental.pallas import tpu_sc as plsc

T = 32768
D = 256
B = 16
NC = 2
NS = 16
NW = NC * NS
SC_ROWS = 8192
TC_ROWS = T - SC_ROWS
RPW = SC_ROWS // NW
CHUNK = 128
NCH = RPW // CHUNK
L = 16
BLK = 2048

_mesh = plsc.VectorSubcoreMesh(
    core_axis_name="c", subcore_axis_name="s", num_cores=NC, num_subcores=NS)


@functools.partial(
    pl.kernel,
    out_type=jax.ShapeDtypeStruct((NC, B, D), jnp.float32),
    mesh=_mesh,
    scratch_types=[
        pltpu.VMEM((2, CHUNK, D), jnp.float32),
        pltpu.VMEM((RPW,), jnp.int32),
        pltpu.VMEM((B * D,), jnp.float32),
        pltpu.VMEM((NS, D), jnp.float32),
        pltpu.VMEM_SHARED((NS, B * D), jnp.float32),
        pltpu.SemaphoreType.DMA,
        pltpu.SemaphoreType.DMA,
    ],
    compiler_params=pltpu.CompilerParams(needs_layout_passes=False),
)
def _segsum_sc(flat_hbm, segs_hbm, out_hbm, buf, segs_v, acc_v, rows_v,
               shared, lsem0, lsem1):
  cid = lax.axis_index("c")
  sid = lax.axis_index("s")
  wid = sid * NC + cid
  base = wid * RPW
  lsems = (lsem0, lsem1)

  pltpu.sync_copy(segs_hbm.at[wid], segs_v)

  pltpu.async_copy(flat_hbm.at[pl.ds(base, CHUNK)], buf.at[0], lsem0)
  pltpu.async_copy(flat_hbm.at[pl.ds(base + CHUNK, CHUNK)], buf.at[1], lsem1)

  zeros16 = jnp.zeros((L,), jnp.float32)
  for j in range(B * D // L):
    acc_v[pl.ds(j * L, L)] = zeros16

  lane = lax.iota(jnp.int32, L)
  cols = [jnp.int32(j * L) + lane for j in range(D // L)]

  def accumulate(bufb, g):
    def group_step(g2, carry):
      row0 = g2 * L
      segs16 = segs_v[pl.ds(g * CHUNK + row0, L)]
      mn = jnp.min(segs16)
      mx = jnp.max(segs16)

      @pl.when(mn == mx)
      def _uniform():
        base_idx = jnp.broadcast_to(mn, (L,)) * D
        for j in range(D // L):
          v = bufb[row0, pl.ds(j * L, L)]
          for i in range(1, L):
            v = v + bufb[row0 + i, pl.ds(j * L, L)]
          plsc.addupdate_scatter(acc_v, [base_idx + cols[j]], v)

      @pl.when(mn != mx)
      def _mixed():
        for i in range(L):
          row = row0 + i
          seg_b = plsc.load_gather(
              segs_v, [jnp.broadcast_to(g * CHUNK + row, (L,))])
          row_base = seg_b * D
          for j in range(D // L):
            vals = bufb[row, pl.ds(j * L, L)]
            plsc.addupdate_scatter(acc_v, [row_base + cols[j]], vals)
      return carry
    lax.fori_loop(0, CHUNK // L, group_step, 0)

  def chunk_step(it, carry):
    for b in range(2):
      g = it * 2 + b
      row0 = base + (g + 2) * CHUNK
      pltpu.make_async_copy(
          flat_hbm.at[pl.ds(base, CHUNK)], buf.at[b], lsems[b]).wait()
      accumulate(buf.at[b], g)
      @pl.when(g + 2 < NCH)
      def _refill():
        pltpu.async_copy(flat_hbm.at[pl.ds(row0, CHUNK)], buf.at[b], lsems[b])
    return carry

  lax.fori_loop(0, NCH // 2, chunk_step, 0)

  pltpu.sync_copy(acc_v, shared.at[sid])
  plsc.subcore_barrier()

  for k in range(NS):
    pltpu.sync_copy(shared.at[k, pl.ds(sid * D, D)], rows_v.at[k])
  for j in range(D // L):
    sl = pl.ds(j * L, L)
    v = rows_v[0, sl]
    for k in range(1, NS):
      v = v + rows_v[k, sl]
    rows_v[0, sl] = v
  pltpu.sync_copy(rows_v.at[0], out_hbm.at[cid, sid])


def _segsum_tc_body(segs_ref, flat_ref, out_ref):
  @pl.when(pl.program_id(0) == 0)
  def _init():
    out_ref[...] = jnp.zeros_like(out_ref)
  seg_row = segs_ref[0, 0, :]
  onehot = (seg_row[None, :] == lax.iota(jnp.int32, B)[:, None]).astype(
      jnp.float32)
  out_ref[...] += jnp.dot(onehot, flat_ref[...],
                          preferred_element_type=jnp.float32,
                          precision=lax.Precision.HIGHEST)


_SC_BLKS = SC_ROWS // BLK

_segsum_tc = pl.pallas_call(
    _segsum_tc_body,
    grid=(TC_ROWS // BLK,),
    in_specs=[
        pl.BlockSpec((1, 1, BLK), lambda i: (_SC_BLKS + i, 0, 0)),
        pl.BlockSpec((BLK, D), lambda i: (_SC_BLKS + i, 0)),
    ],
    out_specs=pl.BlockSpec((B, D), lambda i: (0, 0)),
    out_shape=jax.ShapeDtypeStruct((B, D), jnp.float32),
)


def kernel(flat, segment_ids):
  segs_sc = segment_ids[:SC_ROWS].reshape(NW, RPW)
  segs_tc = segment_ids.reshape(T // BLK, 1, BLK)
  sc_part = _segsum_sc(flat, segs_sc)
  tc_part = _segsum_tc(segs_tc, flat)
  return sc_part[0] + sc_part[1] + tc_part
